# Initial kernel scaffold; baseline (speedup 1.0000x reference)
import jax
import jax.numpy as jnp
from jax.experimental import pallas as pl


def kernel(x_dyn, x_stat, edge_index, edge_weight, W_ih, W_hh, b_ih, b_hh, W1, b1, W2, b2, W_aqi, b_aqi, W_pm, b_pm):
    raise NotImplementedError("write your pallas kernel here")


# speedup vs baseline: 3.3803x; 3.3803x over previous
import functools

import jax
import jax.numpy as jnp
from jax import lax
from jax.experimental import pallas as pl
from jax.experimental.pallas import tpu as pltpu
from jax.experimental.pallas import tpu_sc as plsc

N = 50000
T = 24
DYN = 16
STAT = 16
E = 800000
H = 64
HF = 32

NSUB = 16
WIN = 512
IR = WIN // 128
NWIN = 100
EPT = WIN * NWIN
EPAD = EPT * NSUB
ER = EPAD // 128
NP = 50048
ACC_ROWS = NP
ZCH = ACC_ROWS // NSUB
OPT = NP // NSUB

_f32 = jnp.float32


def _gru_body(xd_ref, xs_ref, wih_t_ref, whh_t_ref, bih_ref, bhh_ref,
              w1a_ref, w1b_ref, out_ref):
    wih_t = wih_t_ref[...]
    whh_t = whh_t_ref[...]
    bih = bih_ref[...]
    bhh = bhh_ref[...]
    R = xd_ref.shape[0]
    h = jnp.zeros((R, H), _f32)
    for t in range(T):
        x_t = xd_ref[:, t * DYN:(t + 1) * DYN]
        gi = jnp.dot(x_t, wih_t, preferred_element_type=_f32) + bih
        gh = jnp.dot(h, whh_t, preferred_element_type=_f32) + bhh
        r = jax.nn.sigmoid(gi[:, 0:H] + gh[:, 0:H])
        z = jax.nn.sigmoid(gi[:, H:2 * H] + gh[:, H:2 * H])
        n = jnp.tanh(gi[:, 2 * H:3 * H] + r * gh[:, 2 * H:3 * H])
        h = (1.0 - z) * n + z * h
    hw = (jnp.dot(h, w1a_ref[...], preferred_element_type=_f32)
          + jnp.dot(xs_ref[...], w1b_ref[...], preferred_element_type=_f32))
    out_ref[0] = hw[:, 0:HF]
    out_ref[1] = hw[:, HF:2 * HF]


def _gru_call(xd2, x_stat, wih_t, whh_t, bih, bhh, w1a, w1b):
    R = 512
    grid = (pl.cdiv(N, R),)
    return pl.pallas_call(
        _gru_body,
        grid=grid,
        in_specs=[
            pl.BlockSpec((R, T * DYN), lambda i: (i, 0)),
            pl.BlockSpec((R, STAT), lambda i: (i, 0)),
            pl.BlockSpec((DYN, 3 * H), lambda i: (0, 0)),
            pl.BlockSpec((H, 3 * H), lambda i: (0, 0)),
            pl.BlockSpec((1, 3 * H), lambda i: (0, 0)),
            pl.BlockSpec((1, 3 * H), lambda i: (0, 0)),
            pl.BlockSpec((H, H), lambda i: (0, 0)),
            pl.BlockSpec((STAT, H), lambda i: (0, 0)),
        ],
        out_specs=pl.BlockSpec((2, R, HF), lambda i: (0, i, 0)),
        out_shape=jax.ShapeDtypeStruct((2, N, HF), _f32),
    )(xd2, x_stat, wih_t, whh_t, bih, bhh, w1a, w1b)


def _mid_body(a_ref, b_ref, b1a_ref, b1b_ref, w2a_ref, w2b_ref, out_ref):
    ha = jnp.maximum(a_ref[...] + b1a_ref[...], 0.0)
    hb = jnp.maximum(b_ref[...] + b1b_ref[...], 0.0)
    hw = (jnp.dot(ha, w2a_ref[...], preferred_element_type=_f32)
          + jnp.dot(hb, w2b_ref[...], preferred_element_type=_f32))
    out_ref[0] = hw[:, 0:HF]
    out_ref[1] = hw[:, HF:2 * HF]


def _mid_call(a, b, b1a, b1b, w2a, w2b):
    R = 2048
    grid = (pl.cdiv(N, R),)
    return pl.pallas_call(
        _mid_body,
        grid=grid,
        in_specs=[
            pl.BlockSpec((R, HF), lambda i: (i, 0)),
            pl.BlockSpec((R, HF), lambda i: (i, 0)),
            pl.BlockSpec((1, HF), lambda i: (0, 0)),
            pl.BlockSpec((1, HF), lambda i: (0, 0)),
            pl.BlockSpec((HF, H), lambda i: (0, 0)),
            pl.BlockSpec((HF, H), lambda i: (0, 0)),
        ],
        out_specs=pl.BlockSpec((2, R, HF), lambda i: (0, i, 0)),
        out_shape=jax.ShapeDtypeStruct((2, N, HF), _f32),
    )(a, b, b1a, b1b, w2a, w2b)


def _head_body(a_ref, b_ref, b2a_ref, b2b_ref, wha_ref, whb_ref, bh_ref,
               out_ref):
    ha = jnp.maximum(a_ref[...] + b2a_ref[...], 0.0)
    hb = jnp.maximum(b_ref[...] + b2b_ref[...], 0.0)
    out_ref[...] = (jnp.dot(ha, wha_ref[...], preferred_element_type=_f32)
                    + jnp.dot(hb, whb_ref[...], preferred_element_type=_f32)
                    + bh_ref[...])


def _head_call(a, b, b2a, b2b, wha, whb, bh):
    R = 2048
    grid = (pl.cdiv(N, R),)
    return pl.pallas_call(
        _head_body,
        grid=grid,
        in_specs=[
            pl.BlockSpec((R, HF), lambda i: (i, 0)),
            pl.BlockSpec((R, HF), lambda i: (i, 0)),
            pl.BlockSpec((1, HF), lambda i: (0, 0)),
            pl.BlockSpec((1, HF), lambda i: (0, 0)),
            pl.BlockSpec((HF, 2), lambda i: (0, 0)),
            pl.BlockSpec((HF, 2), lambda i: (0, 0)),
            pl.BlockSpec((1, 2), lambda i: (0, 0)),
        ],
        out_specs=pl.BlockSpec((R, 2), lambda i: (i, 0)),
        out_shape=jax.ShapeDtypeStruct((N, 2), _f32),
    )(a, b, b2a, b2b, wha, whb, bh)


def _sc_body(table, srcp, dstp, ewp, zer, out, acc, srcv, dstv, ewv, rows,
             sem):
    c = lax.axis_index("c")
    s = lax.axis_index("s")
    pltpu.sync_copy(zer, acc.at[pl.ds(s * ZCH, ZCH)])
    plsc.subcore_barrier()
    cidx = lax.iota(jnp.int32, 16)

    def window(w, carry):
        ebase = s * EPT + w * WIN
        rbase = s * (NWIN * IR) + w * IR
        pltpu.sync_copy(srcp.at[pl.ds(c * ER + rbase, IR)], srcv)
        pltpu.sync_copy(dstp.at[pl.ds(rbase, IR)], dstv)
        pltpu.sync_copy(ewp.at[pl.ds(ebase, WIN)], ewv)
        handles = []
        for j in range(IR):
            handles.append(
                pltpu.async_copy(table.at[srcv.at[j]],
                                 rows.at[pl.ds(j * 128, 128)], sem))
        for h_ in handles:
            h_.wait()

        def scale(g, carry2):
            for u in range(4):
                e = g * 4 + u
                re = jnp.full((16,), e, jnp.int32)
                bb = plsc.load_gather(ewv, [re])
                v0 = plsc.load_gather(rows, [re, cidx])
                v1 = plsc.load_gather(rows, [re, cidx + 16])
                plsc.store_scatter(rows, [re, cidx], v0 * bb)
                plsc.store_scatter(rows, [re, cidx + 16], v1 * bb)
            return carry2

        lax.fori_loop(0, WIN // 4, scale, 0)
        for j in range(IR):
            pltpu.sync_copy(rows.at[pl.ds(j * 128, 128)],
                            acc.at[dstv.at[j]], add=True)
        return carry

    lax.fori_loop(0, NWIN, window, 0)
    plsc.subcore_barrier()
    pltpu.sync_copy(acc.at[pl.ds(s * OPT, OPT)],
                    out.at[pl.ds(c * NP + s * OPT, OPT)])


@functools.cache
def _sc_scatter_kernel():
    return pl.kernel(
        _sc_body,
        out_type=jax.ShapeDtypeStruct((2 * NP, HF), _f32),
        mesh=plsc.VectorSubcoreMesh(core_axis_name="c", subcore_axis_name="s"),
        compiler_params=pltpu.CompilerParams(needs_layout_passes=False,
                                             use_tc_tiling_on_sc=False),
        scratch_types=[
            pltpu.VMEM_SHARED((ACC_ROWS, HF), _f32),
            pltpu.VMEM((IR, 128), jnp.int32),
            pltpu.VMEM((IR, 128), jnp.int32),
            pltpu.VMEM((WIN,), _f32),
            pltpu.VMEM((WIN, HF), _f32),
            pltpu.SemaphoreType.DMA,
        ],
    )


def _sc_scatter(*args):
    return _sc_scatter_kernel()(*args)


def kernel(x_dyn, x_stat, edge_index, edge_weight, W_ih, W_hh, b_ih, b_hh,
           W1, b1, W2, b2, W_aqi, b_aqi, W_pm, b_pm):
    src = edge_index[0].astype(jnp.int32)
    dst = edge_index[1].astype(jnp.int32)
    pad = EPAD - E
    pidx = jnp.arange(pad, dtype=jnp.int32) % 16
    src_p = jnp.concatenate([src, pidx])
    srcp = jnp.concatenate([src_p, src_p + N]).reshape(2 * ER, 128)
    dstp = jnp.concatenate([dst, N + pidx]).reshape(ER, 128)
    ewp = jnp.concatenate([edge_weight.astype(_f32),
                           jnp.zeros((pad,), _f32)])
    zer = jnp.zeros((ZCH, HF), _f32)

    xd2 = x_dyn.reshape(N, T * DYN)
    hw1 = _gru_call(xd2, x_stat, W_ih.T, W_hh.T, b_ih.reshape(1, -1),
                    b_hh.reshape(1, -1), W1[:H], W1[H:])
    agg1 = _sc_scatter(hw1.reshape(2 * N, HF), srcp, dstp, ewp, zer)

    hw2 = _mid_call(agg1[:N], agg1[NP:NP + N], b1[:HF].reshape(1, HF),
                    b1[HF:].reshape(1, HF), W2[:HF], W2[HF:])
    agg2 = _sc_scatter(hw2.reshape(2 * N, HF), srcp, dstp, ewp, zer)

    wh = jnp.concatenate([W_aqi, W_pm], axis=1)
    bh = jnp.stack([b_aqi[0], b_pm[0]]).reshape(1, 2)
    res = _head_call(agg2[:N], agg2[NP:NP + N], b2[:HF].reshape(1, HF),
                     b2[HF:].reshape(1, HF), wh[:HF], wh[HF:], bh)
    return (res[:, 0], res[:, 1])
